# baseline (speedup 1.0000x reference)
import functools

import jax
import jax.numpy as jnp
import numpy as np
from jax import lax
from jax.experimental import pallas as pl
from jax.experimental.pallas import tpu as pltpu
from jax.experimental.pallas import tpu_sc as plsc

_BATCH = 4096
_NROWS = 100000
_DIM = 128
_NW = 32
_RPW = _BATCH // _NW
_NROLE = 16
_PAD = 1024


def _tf2x32(k1, k2, x0, x1):
    rotations = ((13, 15, 26, 6), (17, 29, 16, 24))
    ks = (np.uint32(k1), np.uint32(k2),
          np.uint32(k1) ^ np.uint32(k2) ^ np.uint32(0x1BD11BDA))
    x0 = x0.astype(np.uint32) + ks[0]
    x1 = x1.astype(np.uint32) + ks[1]
    with np.errstate(over="ignore"):
        for d in range(5):
            for r in rotations[d % 2]:
                x0 = x0 + x1
                x1 = (x1 << np.uint32(r)) | (x1 >> np.uint32(32 - r))
                x1 = x1 ^ x0
            x0 = x0 + ks[(d + 1) % 3]
            x1 = x1 + ks[(d + 2) % 3] + np.uint32(d + 1)
    return x0, x1


def _sample_indices_np(seed, batch, maxval):
    f0, f1 = _tf2x32(np.uint32(0), np.uint32(1),
                     np.uint32([0]), np.uint32([seed]))
    s0, s1 = _tf2x32(f0[0], f1[0], np.uint32([0, 0]), np.uint32([0, 1]))
    ar = np.arange(batch, dtype=np.uint32)
    zr = np.zeros(batch, dtype=np.uint32)
    o0, o1 = _tf2x32(s0[0], s1[0], zr, ar)
    y = o0 ^ o1
    o0, o1 = _tf2x32(s0[1], s1[1], zr, ar)
    z = o0 ^ o1
    span = np.uint32(maxval)
    with np.errstate(over="ignore"):
        mult = (np.uint32(65536 % maxval) * np.uint32(65536 % maxval)) % span
        b = ((y % span) * mult + (z % span)) % span
    return b.astype(np.int32)


@functools.lru_cache(maxsize=None)
def _flat_offsets():
    s = [_sample_indices_np(i, _BATCH, _NROWS) for i in range(6)]
    rows = [
        s[0] * 2 + 0, s[0] * 2 + 1,
        s[1] * 3 + 0, s[1] * 3 + 1, s[1] * 3 + 2,
        s[2] * 3 + 0, s[2] * 3 + 1, s[2] * 3 + 2,
        s[3] * 3 + 0, s[3] * 3 + 1, s[3] * 3 + 2,
        s[4] * 2 + 0, s[4] * 2 + 1,
        s[5] * 3 + 0, s[5] * 3 + 1, s[5] * 3 + 2,
    ]
    f = np.stack(rows).astype(np.int32)
    return (f.reshape(_NROLE, _NW, _RPW).transpose(1, 0, 2)
            .reshape(_NW, _NROLE * _RPW).copy())


_ROLE_TAB = (0, 0, 1, 1, 1, 2, 2, 2, 3, 3, 3, 4, 4, 5, 5, 5)
_CLASS_ROLES = (0, 1, 2, 3, 4, 5, 7, 9, 10, 11, 12, 13, 15)
_REL_ROLES = (6, 8, 14)
_CI = {r: i for i, r in enumerate(_CLASS_ROLES)}
_RI = {r: i for i, r in enumerate(_REL_ROLES)}
_COMBOS = (
    ("G", 2, 3), ("G", 2, 4), ("G", 3, 4),
    ("G", 5, 7), ("C", 5, 6), ("C", 7, 6),
    ("G", 9, 10), ("C", 9, 8), ("C", 10, 8),
    ("G", 11, 12),
    ("G", 13, 15), ("C", 13, 14), ("C", 15, 14),
)


def _tc1_body(xs_ref, xst_ref, relt_ref,
              g_ref, c_ref, nr_ref, dg_ref):
    x = xs_ref[...]
    g = jnp.dot(x, xst_ref[...], preferred_element_type=jnp.float32)
    g_ref[...] = g
    c_ref[...] = jnp.dot(x, relt_ref[...], preferred_element_type=jnp.float32)
    rt = relt_ref[...]
    nr_ref[...] = jnp.sum(rt * rt, axis=0, keepdims=True)
    eye = (lax.broadcasted_iota(jnp.int32, (_PAD, _PAD), 0)
           == lax.broadcasted_iota(jnp.int32, (_PAD, _PAD), 1))
    dg_ref[...] = jnp.sum(jnp.where(eye, g, 0.0), axis=1, keepdims=True).T


def _sc_body(g_h, c_h, nr_h, dg_h, rd_h, xs_h,
             nf1_h, nf2_h, nf3_h, nf4_h, dj_h, neg_h, fidx_h,
             sva_h, rows_h,
             fv, cid, gidx, sva, buf0, buf1,
             s0, s1, s2, s3, s4, s5, s6):
    cidx = lax.axis_index("c")
    sidx = lax.axis_index("s")
    wid = sidx * 2 + cidx
    base = wid * _RPW
    nf_tabs = (nf1_h, nf2_h, nf3_h, nf4_h, dj_h, neg_h)
    sems = (s0, s1, s2, s3)

    pltpu.async_copy(fidx_h.at[wid], fv, s4).wait()

    descs = []
    for r in range(_NROLE):
        if r >= 4:
            descs[r - 4].wait()
        descs.append(pltpu.async_copy(
            nf_tabs[_ROLE_TAB[r]].at[fv.at[pl.ds(r * _RPW, _RPW)]],
            cid.at[r, pl.ds(0, _RPW)], sems[r % 4]))
    for d in descs[-4:]:
        d.wait()

    rg0 = pltpu.async_copy(xs_h.at[cid.at[0, pl.ds(0, _RPW)]], buf0, s4)
    rg1 = pltpu.async_copy(xs_h.at[cid.at[1, pl.ds(0, _RPW)]], buf1, s5)

    for n, (_, ri, rj) in enumerate(_COMBOS):
        for k in range(_RPW // 16):
            vi = cid[ri, pl.ds(k * 16, 16)]
            vj = cid[rj, pl.ds(k * 16, 16)]
            gidx[n, pl.ds(k * 16, 16)] = vi * _PAD + vj

    gd = []

    def q(table, idx_ref, dst):
        if len(gd) >= 4:
            gd[len(gd) - 4].wait()
        gd.append(pltpu.async_copy(table.at[idx_ref], dst,
                                   sems[len(gd) % 4]))

    for role in _CLASS_ROLES:
        q(dg_h, cid.at[role, pl.ds(0, _RPW)],
          sva.at[_CI[role], pl.ds(0, _RPW)])
    for role in _CLASS_ROLES:
        q(rd_h, cid.at[role, pl.ds(0, _RPW)],
          sva.at[13 + _CI[role], pl.ds(0, _RPW)])
    for role in _REL_ROLES:
        q(nr_h, cid.at[role, pl.ds(0, _RPW)],
          sva.at[26 + _RI[role], pl.ds(0, _RPW)])
    for n, (tab, _, _) in enumerate(_COMBOS):
        q(g_h if tab == "G" else c_h, gidx.at[n, pl.ds(0, _RPW)],
          sva.at[29 + n, pl.ds(0, _RPW)])
    for d in gd[-4:]:
        d.wait()

    rg0.wait()
    rg1.wait()
    o0 = pltpu.async_copy(buf0, rows_h.at[0, pl.ds(base, _RPW)], s4)
    o1 = pltpu.async_copy(buf1, rows_h.at[1, pl.ds(base, _RPW)], s5)
    o2 = pltpu.async_copy(sva, sva_h.at[wid], s6)
    o0.wait()
    o1.wait()
    o2.wait()


def _tc2_body(sva_ref, rows_ref, out_ref):
    relu = jax.nn.relu

    def n_of(role):
        return sva_ref[:, _CI[role], :]

    def rad(role):
        return jnp.abs(sva_ref[:, 13 + _CI[role], :])

    def nr_of(role):
        return sva_ref[:, 26 + _RI[role], :]

    def combo(n):
        return sva_ref[:, 29 + n, :]

    def reg(n):
        return jnp.abs(jnp.sqrt(n) - 1.0)

    def dist(arg):
        return jnp.sqrt(jnp.maximum(arg, 0.0))

    total = jnp.float32(0.0)

    a = rows_ref[0].reshape(_NW, _RPW, _DIM)
    b = rows_ref[1].reshape(_NW, _RPW, _DIM)
    ra, rb = rad(0), rad(1)
    e = relu(jnp.abs(a - b) + (ra - rb)[:, :, None])
    total += jnp.sum(jnp.sum(e, axis=-1) / _DIM + reg(n_of(0)) + reg(n_of(1)))

    na, nb, nc = n_of(2), n_of(3), n_of(4)
    ra, rb, rc = rad(2), rad(3), rad(4)
    dab = dist(na + nb - 2.0 * combo(0))
    dac = dist(na + nc - 2.0 * combo(1))
    dbc = dist(nb + nc - 2.0 * combo(2))
    total += jnp.sum(relu(dab - (ra + rb)) + relu(dac - ra)
                     + relu(dbc - rb) + relu(jnp.minimum(ra, rb) - rc)
                     + reg(na) + reg(nb) + reg(nc))

    na, nb, nr = n_of(5), n_of(7), nr_of(6)
    ra, rb = rad(5), rad(7)
    euc = dist(na + nb + nr - 2.0 * combo(3) + 2.0 * combo(4)
               - 2.0 * combo(5))
    total += jnp.sum(relu(euc + ra - rb) + reg(na) + reg(nb))

    na, nb, nr = n_of(9), n_of(10), nr_of(8)
    ra, rb = rad(9), rad(10)
    euc = dist(na + nb + nr - 2.0 * combo(6) - 2.0 * combo(7)
               + 2.0 * combo(8))
    total += jnp.sum(relu(euc - (ra + rb)) + reg(na) + reg(nb))

    na, nb = n_of(11), n_of(12)
    ra, rb = rad(11), rad(12)
    euc = dist(na + nb - 2.0 * combo(9))
    total += jnp.sum(relu(ra + rb - euc) + reg(na) + reg(nb))

    na, nb, nr = n_of(13), n_of(15), nr_of(14)
    ra, rb = rad(13), rad(15)
    euc = dist(na + nb + nr - 2.0 * combo(10) + 2.0 * combo(11)
               - 2.0 * combo(12))
    total += jnp.sum((ra + rb - euc) + reg(na) + reg(nb))

    out_ref[0, 0] = total / _BATCH


def kernel(class_emb, rel_emb, nf1, nf2, nf3, nf4, disjoint, nf3_neg):
    class_emb = class_emb.astype(jnp.float32)
    xs = class_emb[:, :_DIM]
    rad = class_emb[:, _DIM]
    rel = rel_emb.astype(jnp.float32)
    nfs = [a.astype(jnp.int32).reshape(-1)
           for a in (nf1, nf2, nf3, nf4, disjoint, nf3_neg)]
    fidx = jnp.asarray(_flat_offsets())

    pad = jnp.zeros((_PAD - xs.shape[0], _DIM), jnp.float32)
    xs_p = jnp.concatenate([xs, pad], axis=0)
    rel_p = jnp.concatenate([rel, pad], axis=0)
    rad_p = jnp.concatenate([rad, jnp.zeros((_PAD - rad.shape[0],),
                                            jnp.float32)]).reshape(1, _PAD)

    gram, cross, nrm, diag = pl.pallas_call(
        _tc1_body,
        out_shape=[
            jax.ShapeDtypeStruct((_PAD, _PAD), jnp.float32),
            jax.ShapeDtypeStruct((_PAD, _PAD), jnp.float32),
            jax.ShapeDtypeStruct((1, _PAD), jnp.float32),
            jax.ShapeDtypeStruct((1, _PAD), jnp.float32),
        ],
    )(xs_p, xs_p.T, rel_p.T)

    mesh = plsc.VectorSubcoreMesh(
        core_axis_name="c", subcore_axis_name="s", num_cores=2,
        num_subcores=16)
    sc_run = pl.kernel(
        _sc_body,
        out_type=[
            jax.ShapeDtypeStruct((_NW, 42, _RPW), jnp.float32),
            jax.ShapeDtypeStruct((2, _BATCH, _DIM), jnp.float32),
        ],
        mesh=mesh,
        scratch_types=[
            pltpu.VMEM((_NROLE * _RPW,), jnp.int32),
            pltpu.VMEM((_NROLE, _RPW), jnp.int32),
            pltpu.VMEM((13, _RPW), jnp.int32),
            pltpu.VMEM((42, _RPW), jnp.float32),
            pltpu.VMEM((_RPW, _DIM), jnp.float32),
            pltpu.VMEM((_RPW, _DIM), jnp.float32),
            pltpu.SemaphoreType.DMA,
            pltpu.SemaphoreType.DMA,
            pltpu.SemaphoreType.DMA,
            pltpu.SemaphoreType.DMA,
            pltpu.SemaphoreType.DMA,
            pltpu.SemaphoreType.DMA,
            pltpu.SemaphoreType.DMA,
        ],
    )
    sva, rows = sc_run(gram.reshape(-1), cross.reshape(-1),
                       nrm.reshape(-1), diag.reshape(-1),
                       rad_p.reshape(-1), xs, *nfs, fidx)

    total = pl.pallas_call(
        _tc2_body,
        out_shape=jax.ShapeDtypeStruct((1, 1), jnp.float32),
        out_specs=pl.BlockSpec(memory_space=pltpu.SMEM),
    )(sva, rows)
    return total[0, 0]
